# baseline (speedup 1.0000x reference)
import functools

import jax
import jax.numpy as jnp
from jax import lax
from jax.experimental import pallas as pl
from jax.experimental.pallas import tpu as pltpu
from jax.experimental.pallas import tpu_sc as plsc

_N = 50000
_E = 800000
_R = 16
_L = 2
_H = 3 * _R
_DEMB = 16
_NSC = 16
_DROT = 2 * _L * _R
_DOUT = 2 * _R
_DT = 128

_NC = 2
_NS = 16
_NW = _NC * _NS

_CHUNK = 128
_EH = _E // 2
_ROWS_PER_TILE = _N // _NS

_mesh = plsc.VectorSubcoreMesh(core_axis_name="c", subcore_axis_name="s")


def _make_gather(ne):
    chunks = ne // _CHUNK
    iters = -(-chunks // _NW)
    iters += iters % 2

    @functools.partial(
        pl.kernel,
        out_type=jax.ShapeDtypeStruct((ne, _DT), jnp.float32),
        mesh=_mesh,
        scratch_types=[
            pltpu.VMEM((2, _CHUNK), jnp.int32),
            pltpu.VMEM((2, _CHUNK, _DT), jnp.float32),
            pltpu.SemaphoreType.DMA,
            pltpu.SemaphoreType.DMA,
        ],
    )
    def _gather(row_hbm, tbl_hbm, out_hbm, idx_v, rows_v, sem0, sem1):
        wid = lax.axis_index("s") * _NC + lax.axis_index("c")
        sems = (sem0, sem1)

        def start(b, cid):
            pltpu.sync_copy(row_hbm.at[pl.ds(cid * _CHUNK, _CHUNK)],
                            idx_v.at[b])
            pltpu.async_copy(tbl_hbm.at[idx_v.at[b]], rows_v.at[b], sems[b])

        def wait(b):
            pltpu.make_async_copy(tbl_hbm.at[idx_v.at[b]], rows_v.at[b],
                                  sems[b]).wait()

        start(0, wid)

        def outer(t, carry):
            for b in (0, 1):
                j = 2 * t + b
                cidn = (j + 1) * _NW + wid

                @pl.when(cidn < chunks)
                def _():
                    start(1 - b, cidn)

                cid = j * _NW + wid

                @pl.when(cid < chunks)
                def _():
                    wait(b)
                    pltpu.sync_copy(rows_v.at[b],
                                    out_hbm.at[pl.ds(cid * _CHUNK, _CHUNK)])

            return carry

        lax.fori_loop(0, iters // 2, outer, 0)

    return _gather


def _make_scatter(ne):
    chunks = ne // _CHUNK
    iters = -(-(-(-chunks // 2)) // _NS)
    iters += iters % 2

    @functools.partial(
        pl.kernel,
        out_type=jax.ShapeDtypeStruct((_NC, _N, _DOUT), jnp.float32),
        mesh=_mesh,
        scratch_types=[
            pltpu.VMEM((2, _CHUNK), jnp.int32),
            pltpu.VMEM((2, _CHUNK, _DOUT), jnp.float32),
            pltpu.VMEM_SHARED((_N, _DOUT), jnp.float32),
            pltpu.SemaphoreType.DMA,
            pltpu.SemaphoreType.DMA,
        ],
        compiler_params=pltpu.CompilerParams(use_tc_tiling_on_sc=False),
    )
    def _scatter(col_hbm, xout_hbm, zero_hbm, part_hbm, idx_v, rows_v, acc,
                 sem0, sem1):
        c = lax.axis_index("c")
        s = lax.axis_index("s")
        sems = (sem0, sem1)
        r0 = s * _ROWS_PER_TILE
        pltpu.sync_copy(zero_hbm.at[pl.ds(r0, _ROWS_PER_TILE)],
                        acc.at[pl.ds(r0, _ROWS_PER_TILE)])
        plsc.subcore_barrier()

        def start(b, gcid):
            base = gcid * _CHUNK
            pltpu.sync_copy(col_hbm.at[pl.ds(base, _CHUNK)], idx_v.at[b])
            pltpu.async_copy(xout_hbm.at[pl.ds(base, _CHUNK)], rows_v.at[b],
                             sems[b])

        def wait(b, gcid):
            base = gcid * _CHUNK
            pltpu.make_async_copy(xout_hbm.at[pl.ds(base, _CHUNK)],
                                  rows_v.at[b], sems[b]).wait()

        start(0, s * _NC + c)

        def outer(t, carry):
            for b in (0, 1):
                j = 2 * t + b
                gcidn = ((j + 1) * _NS + s) * _NC + c

                @pl.when(gcidn < chunks)
                def _():
                    start(1 - b, gcidn)

                gcid = (j * _NS + s) * _NC + c

                @pl.when(gcid < chunks)
                def _():
                    wait(b, gcid)
                    pltpu.sync_copy(rows_v.at[b], acc.at[idx_v.at[b]],
                                    add=True)

            return carry

        lax.fori_loop(0, iters // 2, outer, 0)
        plsc.subcore_barrier()
        pltpu.sync_copy(acc.at[pl.ds(r0, _ROWS_PER_TILE)],
                        part_hbm.at[c, pl.ds(r0, _ROWS_PER_TILE)])

    return _scatter


_BE = 4000


def _mlp_body(gthr_ref, demb_ref, wxd_ref, b_ref, mrot_ref, m2_ref, out_ref):
    gt = gthr_ref[...].astype(jnp.bfloat16)
    xr = gt[:, :_DROT]
    sc = jnp.concatenate(
        [gt[:, _DROT:_DROT + _NSC],
         demb_ref[...].astype(jnp.bfloat16)], axis=1)
    z = (jnp.dot(sc, wxd_ref[...], preferred_element_type=jnp.float32)
         + b_ref[...])
    g = z * jax.nn.sigmoid(z)
    h2 = jnp.dot(xr, mrot_ref[...], preferred_element_type=jnp.float32)
    res = jnp.dot((h2 * g).astype(jnp.bfloat16), m2_ref[...],
                  preferred_element_type=jnp.float32)
    q = _BE // 4
    out_ref[...] = jnp.concatenate(
        [res[0:q], res[q:2 * q], res[2 * q:3 * q], res[3 * q:4 * q]], axis=1)


def _make_mlp(ne, demb_off):
    return pl.pallas_call(
        _mlp_body,
        grid=(ne // _BE,),
        in_specs=[
            pl.BlockSpec((_BE, _DT), lambda i: (i, 0)),
            pl.BlockSpec((_BE, _DEMB), lambda i: (i + demb_off, 0)),
            pl.BlockSpec((_NSC + _DEMB, 4 * _H), lambda i: (0, 0)),
            pl.BlockSpec((1, 4 * _H), lambda i: (0, 0)),
            pl.BlockSpec((_DROT, 4 * _H), lambda i: (0, 0)),
            pl.BlockSpec((4 * _H, _DOUT), lambda i: (0, 0)),
        ],
        out_specs=pl.BlockSpec((_BE // 4, 128), lambda i: (i, 0)),
        out_shape=jax.ShapeDtypeStruct((ne // 4, 128), jnp.float32),
    )


def _add4_body(a_ref, b_ref, c_ref, d_ref, o_ref):
    o_ref[...] = ((a_ref[...] + b_ref[...]) + (c_ref[...] + d_ref[...]))


_BN = 2000

_combine = pl.pallas_call(
    _add4_body,
    grid=(_N // _BN,),
    in_specs=[pl.BlockSpec((_BN, _DOUT), lambda i: (i, 0))] * 4,
    out_specs=pl.BlockSpec((_BN, _DOUT), lambda i: (i, 0)),
    out_shape=jax.ShapeDtypeStruct((_N, _DOUT), jnp.float32),
)

_gather_h = _make_gather(_EH)
_scatter_h = _make_scatter(_EH)
_mlp_h0 = _make_mlp(_EH, 0)
_mlp_h1 = _make_mlp(_EH, _EH // _BE)


def _perm_col(col_h):
    q = _BE // 4
    return col_h.reshape(-1, 4, q).transpose(0, 2, 1).reshape(col_h.shape)


def kernel(x_scalar, x_rot, edge_index, distance_embedding, rot,
           W_rot, W_s1, b_s1, W_out):
    del rot
    row = edge_index[0]
    col = edge_index[1]
    tbl = jnp.concatenate(
        [x_rot.reshape(_N, _DROT), x_scalar,
         jnp.zeros((_N, _DT - _DROT - _NSC), jnp.float32)], axis=1)

    mrot = jnp.kron(W_rot, jnp.eye(4, dtype=jnp.float32))
    ws1e = jnp.broadcast_to(
        W_s1.reshape(_DEMB + _NSC, _H, _L, 1),
        (_DEMB + _NSC, _H, _L, 2)).reshape(_DEMB + _NSC, 4 * _H)
    be = jnp.broadcast_to(b_s1.reshape(_H, _L, 1),
                          (_H, _L, 2)).reshape(1, 4 * _H)
    m2 = jnp.einsum("hlo,cd->hlcod", W_out,
                    jnp.eye(2, dtype=jnp.float32)).reshape(4 * _H, _DOUT)
    wxd = jnp.concatenate([ws1e[_DEMB:], ws1e[:_DEMB]], axis=0)
    wxd16 = wxd.astype(jnp.bfloat16)
    mrot16 = mrot.astype(jnp.bfloat16)
    m216 = m2.astype(jnp.bfloat16)

    zeros = jnp.zeros((_N, _DOUT), jnp.float32)

    gthr0 = _gather_h(row[:_EH], tbl)
    gthr1 = _gather_h(row[_EH:], tbl)
    xout0 = _mlp_h0(gthr0, distance_embedding, wxd16, be, mrot16, m216)
    xout1 = _mlp_h1(gthr1, distance_embedding, wxd16, be, mrot16, m216)
    parts0 = _scatter_h(_perm_col(col[:_EH]), xout0.reshape(_EH, _DOUT),
                        zeros)
    parts1 = _scatter_h(_perm_col(col[_EH:]), xout1.reshape(_EH, _DOUT),
                        zeros)

    mess = _combine(parts0[0], parts0[1], parts1[0], parts1[1])
    return (x_scalar, mess.reshape(_N, _R, 2))


# speedup vs baseline: 1.0097x; 1.0097x over previous
import functools

import jax
import jax.numpy as jnp
from jax import lax
from jax.experimental import pallas as pl
from jax.experimental.pallas import tpu as pltpu
from jax.experimental.pallas import tpu_sc as plsc

_N = 50000
_E = 800000
_R = 16
_L = 2
_H = 3 * _R
_DEMB = 16
_NSC = 16
_DROT = 2 * _L * _R
_DOUT = 2 * _R
_DT = 128

_NC = 2
_NS = 16
_NW = _NC * _NS

_CHUNK = 128
_EH = _E // 2
_ROWS_PER_TILE = _N // _NS

_mesh = plsc.VectorSubcoreMesh(core_axis_name="c", subcore_axis_name="s")


def _make_gather(ne):
    chunks = ne // _CHUNK
    iters = -(-chunks // _NW)
    iters += iters % 2

    @functools.partial(
        pl.kernel,
        out_type=jax.ShapeDtypeStruct((ne, _DT), jnp.float32),
        mesh=_mesh,
        scratch_types=[
            pltpu.VMEM((2, _CHUNK), jnp.int32),
            pltpu.VMEM((2, _CHUNK, _DT), jnp.float32),
            pltpu.SemaphoreType.DMA,
            pltpu.SemaphoreType.DMA,
        ],
    )
    def _gather(row_hbm, tbl_hbm, out_hbm, idx_v, rows_v, sem0, sem1):
        wid = lax.axis_index("s") * _NC + lax.axis_index("c")
        sems = (sem0, sem1)

        def start(b, cid):
            pltpu.sync_copy(row_hbm.at[pl.ds(cid * _CHUNK, _CHUNK)],
                            idx_v.at[b])
            pltpu.async_copy(tbl_hbm.at[idx_v.at[b]], rows_v.at[b], sems[b])

        def wait(b):
            pltpu.make_async_copy(tbl_hbm.at[idx_v.at[b]], rows_v.at[b],
                                  sems[b]).wait()

        start(0, wid)

        def outer(t, carry):
            for b in (0, 1):
                j = 2 * t + b
                cidn = (j + 1) * _NW + wid

                @pl.when(cidn < chunks)
                def _():
                    start(1 - b, cidn)

                cid = j * _NW + wid

                @pl.when(cid < chunks)
                def _():
                    wait(b)
                    pltpu.sync_copy(rows_v.at[b],
                                    out_hbm.at[pl.ds(cid * _CHUNK, _CHUNK)])

            return carry

        lax.fori_loop(0, iters // 2, outer, 0)

    return _gather


def _make_scatter(ne):
    chunks = ne // _CHUNK
    iters = -(-(-(-chunks // 2)) // _NS)
    iters += iters % 2

    @functools.partial(
        pl.kernel,
        out_type=jax.ShapeDtypeStruct((_NC, _N, _DOUT), jnp.float32),
        mesh=_mesh,
        scratch_types=[
            pltpu.VMEM((2, _CHUNK), jnp.int32),
            pltpu.VMEM((2, _CHUNK, _DOUT), jnp.float32),
            pltpu.VMEM_SHARED((_N, _DOUT), jnp.float32),
            pltpu.SemaphoreType.DMA,
            pltpu.SemaphoreType.DMA,
        ],
        compiler_params=pltpu.CompilerParams(use_tc_tiling_on_sc=False),
    )
    def _scatter(col_hbm, xout_hbm, zero_hbm, part_hbm, idx_v, rows_v, acc,
                 sem0, sem1):
        c = lax.axis_index("c")
        s = lax.axis_index("s")
        sems = (sem0, sem1)
        r0 = s * _ROWS_PER_TILE
        pltpu.sync_copy(zero_hbm.at[pl.ds(r0, _ROWS_PER_TILE)],
                        acc.at[pl.ds(r0, _ROWS_PER_TILE)])
        plsc.subcore_barrier()

        def start(b, gcid):
            base = gcid * _CHUNK
            pltpu.sync_copy(col_hbm.at[pl.ds(base, _CHUNK)], idx_v.at[b])
            pltpu.async_copy(xout_hbm.at[pl.ds(base, _CHUNK)], rows_v.at[b],
                             sems[b])

        def wait(b, gcid):
            base = gcid * _CHUNK
            pltpu.make_async_copy(xout_hbm.at[pl.ds(base, _CHUNK)],
                                  rows_v.at[b], sems[b]).wait()

        start(0, s * _NC + c)

        def outer(t, carry):
            for b in (0, 1):
                j = 2 * t + b
                gcidn = ((j + 1) * _NS + s) * _NC + c

                @pl.when(gcidn < chunks)
                def _():
                    start(1 - b, gcidn)

                gcid = (j * _NS + s) * _NC + c

                @pl.when(gcid < chunks)
                def _():
                    wait(b, gcid)
                    pltpu.sync_copy(rows_v.at[b], acc.at[idx_v.at[b]],
                                    add=True)

            return carry

        lax.fori_loop(0, iters // 2, outer, 0)
        plsc.subcore_barrier()
        pltpu.sync_copy(acc.at[pl.ds(r0, _ROWS_PER_TILE)],
                        part_hbm.at[c, pl.ds(r0, _ROWS_PER_TILE)])

    return _scatter


_BE = 4000


def _mlp_body(gthr_ref, demb_ref, wxd_ref, b_ref, mrot_ref, m2_ref, out_ref):
    gt = gthr_ref[...].astype(jnp.bfloat16)
    xr = gt[:, :_DROT]
    sc = jnp.concatenate(
        [gt[:, _DROT:_DROT + _NSC], demb_ref[...]], axis=1)
    z = (jnp.dot(sc, wxd_ref[...], preferred_element_type=jnp.float32)
         + b_ref[...])
    g = z * jax.nn.sigmoid(z)
    h2 = jnp.dot(xr, mrot_ref[...], preferred_element_type=jnp.float32)
    res = jnp.dot((h2 * g).astype(jnp.bfloat16), m2_ref[...],
                  preferred_element_type=jnp.float32)
    q = _BE // 4
    out_ref[...] = jnp.concatenate(
        [res[0:q], res[q:2 * q], res[2 * q:3 * q], res[3 * q:4 * q]], axis=1)


def _make_mlp(ne):
    return pl.pallas_call(
        _mlp_body,
        grid=(ne // _BE,),
        in_specs=[
            pl.BlockSpec((_BE, _DT), lambda i: (i, 0)),
            pl.BlockSpec((_BE, _DEMB), lambda i: (i, 0)),
            pl.BlockSpec((_NSC + _DEMB, 4 * _H), lambda i: (0, 0)),
            pl.BlockSpec((1, 4 * _H), lambda i: (0, 0)),
            pl.BlockSpec((_DROT, 4 * _H), lambda i: (0, 0)),
            pl.BlockSpec((4 * _H, _DOUT), lambda i: (0, 0)),
        ],
        out_specs=pl.BlockSpec((_BE // 4, 128), lambda i: (i, 0)),
        out_shape=jax.ShapeDtypeStruct((ne // 4, 128), jnp.float32),
    )


def _add4_body(a_ref, b_ref, c_ref, d_ref, o_ref):
    o_ref[...] = ((a_ref[...] + b_ref[...]) + (c_ref[...] + d_ref[...]))


_BN = 2000

_combine = pl.pallas_call(
    _add4_body,
    grid=(_N // _BN,),
    in_specs=[pl.BlockSpec((_BN, _DOUT), lambda i: (i, 0))] * 4,
    out_specs=pl.BlockSpec((_BN, _DOUT), lambda i: (i, 0)),
    out_shape=jax.ShapeDtypeStruct((_N, _DOUT), jnp.float32),
)

_gather_h = _make_gather(_EH)
_scatter_h = _make_scatter(_EH)
_mlp_h = _make_mlp(_EH)


def _perm_col(col_h):
    q = _BE // 4
    return col_h.reshape(-1, 4, q).transpose(0, 2, 1).reshape(col_h.shape)


def kernel(x_scalar, x_rot, edge_index, distance_embedding, rot,
           W_rot, W_s1, b_s1, W_out):
    del rot
    row = edge_index[0]
    col = edge_index[1]
    tbl = jnp.concatenate(
        [x_rot.reshape(_N, _DROT), x_scalar,
         jnp.zeros((_N, _DT - _DROT - _NSC), jnp.float32)], axis=1)

    mrot = jnp.kron(W_rot, jnp.eye(4, dtype=jnp.float32))
    ws1e = jnp.broadcast_to(
        W_s1.reshape(_DEMB + _NSC, _H, _L, 1),
        (_DEMB + _NSC, _H, _L, 2)).reshape(_DEMB + _NSC, 4 * _H)
    be = jnp.broadcast_to(b_s1.reshape(_H, _L, 1),
                          (_H, _L, 2)).reshape(1, 4 * _H)
    m2 = jnp.einsum("hlo,cd->hlcod", W_out,
                    jnp.eye(2, dtype=jnp.float32)).reshape(4 * _H, _DOUT)
    wxd = jnp.concatenate([ws1e[_DEMB:], ws1e[:_DEMB]], axis=0)
    wxd16 = wxd.astype(jnp.bfloat16)
    mrot16 = mrot.astype(jnp.bfloat16)
    m216 = m2.astype(jnp.bfloat16)

    zeros = jnp.zeros((_N, _DOUT), jnp.float32)

    demb16 = distance_embedding.astype(jnp.bfloat16)
    gthr0 = _gather_h(row[:_EH], tbl)
    gthr1 = _gather_h(row[_EH:], tbl)
    xout0 = _mlp_h(gthr0, demb16[:_EH], wxd16, be, mrot16, m216)
    xout1 = _mlp_h(gthr1, demb16[_EH:], wxd16, be, mrot16, m216)
    parts0 = _scatter_h(_perm_col(col[:_EH]), xout0.reshape(_EH, _DOUT),
                        zeros)
    parts1 = _scatter_h(_perm_col(col[_EH:]), xout1.reshape(_EH, _DOUT),
                        zeros)

    mess = _combine(parts0[0], parts0[1], parts1[0], parts1[1])
    return (x_scalar, mess.reshape(_N, _R, 2))
